# Initial kernel scaffold; baseline (speedup 1.0000x reference)
import jax
import jax.numpy as jnp
from jax.experimental import pallas as pl


def kernel(x, edge_index, W1, b1, W2, b2, P1w, P1b, P2w, P2b):
    raise NotImplementedError("write your pallas kernel here")


# speedup vs baseline: 20.7671x; 20.7671x over previous
import functools

import jax
import jax.numpy as jnp
from jax import lax
from jax.experimental import pallas as pl
from jax.experimental.pallas import tpu as pltpu
from jax.experimental.pallas import tpu_sc as plsc

_N = 10000
_E = 320000
_D = 128
_DP = 64
_NPAD = 10368
_EPAD = 331776
_CH = 128
_NCHT = _EPAD // 32 // _CH
_RPT = _NPAD // 16

_NC, _NS = 2, 16


def _mesh():
    return plsc.VectorSubcoreMesh(core_axis_name="c", subcore_axis_name="s")


def _deg_body(dst_hbm, zero_hbm, out_hbm, didx, ones_v, stg, dacc, sem):
    c = lax.axis_index("c")
    s = lax.axis_index("s")
    wid = c * _NS + s
    pltpu.sync_copy(zero_hbm.at[pl.ds(s * _RPT, _RPT)], stg)
    pltpu.sync_copy(stg, dacc.at[pl.ds(s * _RPT, _RPT)])
    pltpu.sync_copy(dst_hbm.at[wid], didx)
    for i in range(_CH // 16):
        ones_v[pl.ds(i * 16, 16)] = jnp.ones((16,), jnp.float32)
    plsc.subcore_barrier()

    def body(j, carry):
        pltpu.sync_copy(ones_v, dacc.at[didx.at[j]], add=True)
        return carry

    lax.fori_loop(0, _NCHT, body, 0)
    plsc.subcore_barrier()
    pltpu.sync_copy(dacc.at[pl.ds(s * _RPT, _RPT)], stg)
    pltpu.sync_copy(stg, out_hbm.at[pl.ds(c * _NPAD + s * _RPT, _RPT)])


@functools.partial(
    pl.kernel,
    mesh=_mesh(),
    out_type=jax.ShapeDtypeStruct((_NC * _NPAD,), jnp.float32),
    scratch_types=[
        pltpu.VMEM((_NCHT, _CH), jnp.int32),
        pltpu.VMEM((_CH,), jnp.float32),
        pltpu.VMEM((_RPT,), jnp.float32),
        pltpu.VMEM_SHARED((_NPAD,), jnp.float32),
        pltpu.SemaphoreType.DMA,
    ],
)
def _deg_sc(dst_hbm, zero_hbm, out_hbm, didx, ones_v, stg, dacc, sem):
    _deg_body(dst_hbm, zero_hbm, out_hbm, didx, ones_v, stg, dacc, sem)


_WBR = 72


def _agg_body(y_hbm, src_hbm, dst_hbm, zero_hbm, out_hbm,
              sidx, didx, rows, stg, acc, sem):
    c = lax.axis_index("c")
    s = lax.axis_index("s")
    wid = c * _NS + s
    pltpu.sync_copy(zero_hbm.at[pl.ds(s * _RPT, _WBR)], stg)
    for k in range(_RPT // _WBR):
        pltpu.sync_copy(stg, acc.at[pl.ds(s * _RPT + k * _WBR, _WBR)])
    pltpu.sync_copy(src_hbm.at[wid], sidx)
    pltpu.sync_copy(dst_hbm.at[wid], didx)
    plsc.subcore_barrier()

    def body(j, carry):
        pltpu.async_copy(y_hbm.at[sidx.at[j]], rows, sem).wait()
        pltpu.sync_copy(rows, acc.at[didx.at[j]], add=True)
        return carry

    lax.fori_loop(0, _NCHT, body, 0)
    plsc.subcore_barrier()
    for k in range(_RPT // _WBR):
        pltpu.sync_copy(acc.at[pl.ds(s * _RPT + k * _WBR, _WBR)], stg)
        pltpu.sync_copy(
            stg, out_hbm.at[pl.ds(c * _NPAD + s * _RPT + k * _WBR, _WBR)])


@functools.partial(
    pl.kernel,
    mesh=_mesh(),
    out_type=jax.ShapeDtypeStruct((_NC * _NPAD, _D), jnp.float32),
    scratch_types=[
        pltpu.VMEM((_NCHT, _CH), jnp.int32),
        pltpu.VMEM((_NCHT, _CH), jnp.int32),
        pltpu.VMEM((_CH, _D), jnp.float32),
        pltpu.VMEM((_WBR, _D), jnp.float32),
        pltpu.VMEM_SHARED((_NPAD, _D), jnp.float32),
        pltpu.SemaphoreType.DMA,
    ],
)
def _agg_sc(y_hbm, src_hbm, dst_hbm, zero_hbm, out_hbm,
            sidx, didx, rows, stg, acc, sem):
    _agg_body(y_hbm, src_hbm, dst_hbm, zero_hbm, out_hbm,
              sidx, didx, rows, stg, acc, sem)


_BR = 3456


def _mm1_body(x_ref, w_ref, d0_ref, d1_ref, y_ref, dinv_ref):
    deg = d0_ref[...] + d1_ref[...]
    dinv = lax.rsqrt(jnp.maximum(deg, 1e-12))
    dinv_ref[...] = dinv
    xw = jnp.dot(x_ref[...], w_ref[...], preferred_element_type=jnp.float32)
    y_ref[...] = xw * dinv


def _tc_mm1(xpad, W1, deg0, deg1):
    return pl.pallas_call(
        _mm1_body,
        grid=(_NPAD // _BR,),
        in_specs=[
            pl.BlockSpec((_BR, _D), lambda i: (i, 0)),
            pl.BlockSpec((_D, _D), lambda i: (0, 0)),
            pl.BlockSpec((_BR, 1), lambda i: (i, 0)),
            pl.BlockSpec((_BR, 1), lambda i: (i, 0)),
        ],
        out_specs=[
            pl.BlockSpec((_BR, _D), lambda i: (i, 0)),
            pl.BlockSpec((_BR, 1), lambda i: (i, 0)),
        ],
        out_shape=[
            jax.ShapeDtypeStruct((_NPAD, _D), jnp.float32),
            jax.ShapeDtypeStruct((_NPAD, 1), jnp.float32),
        ],
    )(xpad, W1, deg0, deg1)


def _mid_body(a0_ref, a1_ref, dinv_ref, b1_ref, w2_ref, y2_ref):
    dinv = dinv_ref[...]
    h1 = jnp.maximum((a0_ref[...] + a1_ref[...]) * dinv + b1_ref[...], 0.0)
    y2_ref[...] = jnp.dot(h1, w2_ref[...],
                          preferred_element_type=jnp.float32) * dinv


def _tc_mid(a0, a1, dinv, b1, W2):
    return pl.pallas_call(
        _mid_body,
        grid=(_NPAD // _BR,),
        in_specs=[
            pl.BlockSpec((_BR, _D), lambda i: (i, 0)),
            pl.BlockSpec((_BR, _D), lambda i: (i, 0)),
            pl.BlockSpec((_BR, 1), lambda i: (i, 0)),
            pl.BlockSpec((1, _D), lambda i: (0, 0)),
            pl.BlockSpec((_D, _D), lambda i: (0, 0)),
        ],
        out_specs=pl.BlockSpec((_BR, _D), lambda i: (i, 0)),
        out_shape=jax.ShapeDtypeStruct((_NPAD, _D), jnp.float32),
    )(a0, a1, dinv, b1, W2)


def _fin_body(a0_ref, a1_ref, dinv_ref, b2_ref, p1w_ref, p1b_ref,
              p2w_ref, p2b_ref, z_ref, h_ref):
    z = (a0_ref[...] + a1_ref[...]) * dinv_ref[...] + b2_ref[...]
    z_ref[...] = z
    t = jnp.maximum(
        jnp.dot(z, p1w_ref[...], preferred_element_type=jnp.float32)
        + p1b_ref[...], 0.0)
    h_ref[...] = jnp.dot(t, p2w_ref[...],
                         preferred_element_type=jnp.float32) + p2b_ref[...]


def _tc_fin(a0, a1, dinv, b2, P1w, P1b, P2w, P2b):
    return pl.pallas_call(
        _fin_body,
        grid=(_NPAD // _BR,),
        in_specs=[
            pl.BlockSpec((_BR, _D), lambda i: (i, 0)),
            pl.BlockSpec((_BR, _D), lambda i: (i, 0)),
            pl.BlockSpec((_BR, 1), lambda i: (i, 0)),
            pl.BlockSpec((1, _D), lambda i: (0, 0)),
            pl.BlockSpec((_D, _D), lambda i: (0, 0)),
            pl.BlockSpec((1, _D), lambda i: (0, 0)),
            pl.BlockSpec((_D, _DP), lambda i: (0, 0)),
            pl.BlockSpec((1, _DP), lambda i: (0, 0)),
        ],
        out_specs=[
            pl.BlockSpec((_BR, _D), lambda i: (i, 0)),
            pl.BlockSpec((_BR, _DP), lambda i: (i, 0)),
        ],
        out_shape=[
            jax.ShapeDtypeStruct((_NPAD, _D), jnp.float32),
            jax.ShapeDtypeStruct((_NPAD, _DP), jnp.float32),
        ],
    )(a0, a1, dinv, b2, P1w, P1b, P2w, P2b)


def kernel(x, edge_index, W1, b1, W2, b2, P1w, P1b, P2w, P2b):
    src = edge_index[0]
    dst = edge_index[1]
    loop = jnp.arange(_N, dtype=jnp.int32)
    npadd = _EPAD - _E - _N
    pad_src = (jnp.arange(npadd, dtype=jnp.int32) * 7) % _N
    pad_dst = _N + jnp.arange(npadd, dtype=jnp.int32) % (_NPAD - _N)
    srcp = jnp.concatenate([src, loop, pad_src]).reshape(32, _NCHT, _CH)
    dstp = jnp.concatenate([dst, loop, pad_dst]).reshape(32, _NCHT, _CH)

    xpad = jnp.zeros((_NPAD, _D), jnp.float32).at[:_N].set(x)
    zeros1 = jnp.zeros((_NPAD,), jnp.float32)
    zeros2 = jnp.zeros((_NPAD, _D), jnp.float32)

    deg = _deg_sc(dstp, zeros1)
    deg0 = deg[:_NPAD].reshape(_NPAD, 1)
    deg1 = deg[_NPAD:].reshape(_NPAD, 1)

    y1, dinv = _tc_mm1(xpad, W1, deg0, deg1)

    agg1 = _agg_sc(y1, srcp, dstp, zeros2)
    y2 = _tc_mid(agg1[:_NPAD], agg1[_NPAD:], dinv, b1.reshape(1, _D), W2)

    agg2 = _agg_sc(y2, srcp, dstp, zeros2)
    z, h = _tc_fin(agg2[:_NPAD], agg2[_NPAD:], dinv, b2.reshape(1, _D),
                   P1w, P1b.reshape(1, _D), P2w, P2b.reshape(1, _DP))
    return (z[:_N], h[:_N])


# speedup vs baseline: 26.0629x; 1.2550x over previous
import functools

import jax
import jax.numpy as jnp
from jax import lax
from jax.experimental import pallas as pl
from jax.experimental.pallas import tpu as pltpu
from jax.experimental.pallas import tpu_sc as plsc

_N = 10000
_E = 320000
_D = 128
_DP = 64
_NPAD = 10240
_EPAD = 331776
_EPW = _EPAD // 32
_CH = 64
_NCHT = _EPW // _CH
_RPT = _NPAD // 16

_NC, _NS = 2, 16


def _mesh():
    return plsc.VectorSubcoreMesh(core_axis_name="c", subcore_axis_name="s")


def _deg_body(dst_hbm, zero_hbm, out_hbm, didx, ones_v, stg, dacc, sem):
    c = lax.axis_index("c")
    s = lax.axis_index("s")
    wid = c * _NS + s
    pltpu.sync_copy(zero_hbm.at[pl.ds(s * _RPT, _RPT)], stg)
    pltpu.sync_copy(stg, dacc.at[pl.ds(s * _RPT, _RPT)])
    pltpu.sync_copy(dst_hbm.at[wid], didx)
    for i in range(_CH // 16):
        ones_v[pl.ds(i * 16, 16)] = jnp.ones((16,), jnp.float32)
    plsc.subcore_barrier()

    def body(j, carry):
        pltpu.sync_copy(ones_v, dacc.at[didx.at[j]], add=True)
        return carry

    lax.fori_loop(0, _NCHT, body, 0)
    plsc.subcore_barrier()
    pltpu.sync_copy(dacc.at[pl.ds(s * _RPT, _RPT)], stg)
    pltpu.sync_copy(stg, out_hbm.at[pl.ds(c * _NPAD + s * _RPT, _RPT)])


@functools.partial(
    pl.kernel,
    mesh=_mesh(),
    out_type=jax.ShapeDtypeStruct((_NC * _NPAD,), jnp.float32),
    scratch_types=[
        pltpu.VMEM((_NCHT, _CH), jnp.int32),
        pltpu.VMEM((_CH,), jnp.float32),
        pltpu.VMEM((_RPT,), jnp.float32),
        pltpu.VMEM_SHARED((_NPAD,), jnp.float32),
        pltpu.SemaphoreType.DMA,
    ],
)
def _deg_sc(dst_hbm, zero_hbm, out_hbm, didx, ones_v, stg, dacc, sem):
    _deg_body(dst_hbm, zero_hbm, out_hbm, didx, ones_v, stg, dacc, sem)


_WBR = 72


def _agg_body(y_hbm, src_hbm, dst_hbm, zero_hbm, out_hbm,
              sidx, didx, r0, r1, acc,
              sg0, sg1, ss0, ss1):
    c = lax.axis_index("c")
    s = lax.axis_index("s")
    wid = c * _NS + s
    pltpu.sync_copy(zero_hbm.at[pl.ds(s * _RPT, _CH)], r0)
    for k in range(_RPT // _CH):
        pltpu.sync_copy(r0, acc.at[pl.ds(s * _RPT + k * _CH, _CH)])
    pltpu.sync_copy(src_hbm.at[pl.ds(wid * _EPW, _EPW)], sidx)
    pltpu.sync_copy(dst_hbm.at[wid], didx)
    plsc.subcore_barrier()

    def sl(j):
        return sidx.at[pl.ds(j * _CH, _CH)]

    def wait_g(buf, sem, j):
        pltpu.make_async_copy(y_hbm.at[sl(j)], buf, sem).wait()

    def wait_s(buf, sem, j):
        pltpu.make_async_copy(buf, acc.at[didx.at[j]], sem).wait()

    pltpu.async_copy(y_hbm.at[sl(0)], r0, sg0)

    def body(jj, carry):
        j = 2 * jj
        @pl.when(jj > 0)
        def _():
            wait_s(r1, ss1, j - 1)
        pltpu.async_copy(y_hbm.at[sl(j + 1)], r1, sg1)
        wait_g(r0, sg0, j)
        pltpu.async_copy(r0, acc.at[didx.at[j]], ss0, add=True)
        wait_s(r0, ss0, j)
        @pl.when(j + 2 < _NCHT)
        def _():
            pltpu.async_copy(y_hbm.at[sl(j + 2)], r0, sg0)
        wait_g(r1, sg1, j + 1)
        pltpu.async_copy(r1, acc.at[didx.at[j + 1]], ss1, add=True)
        return carry

    lax.fori_loop(0, _NCHT // 2, body, 0)
    last = _NCHT - 1
    if _NCHT % 2:
        wait_s(r1, ss1, last - 1)
        wait_g(r0, sg0, last)
        pltpu.async_copy(r0, acc.at[didx.at[last]], ss0, add=True)
        wait_s(r0, ss0, last)
    else:
        wait_s(r1, ss1, last)
    plsc.subcore_barrier()
    for k in range(_RPT // _CH):
        pltpu.sync_copy(acc.at[pl.ds(s * _RPT + k * _CH, _CH)], r0)
        pltpu.sync_copy(
            r0, out_hbm.at[pl.ds(c * _NPAD + s * _RPT + k * _CH, _CH)])


@functools.partial(
    pl.kernel,
    mesh=_mesh(),
    out_type=jax.ShapeDtypeStruct((_NC * _NPAD, _D), jnp.float32),
    scratch_types=[
        pltpu.VMEM((_EPW,), jnp.int32),
        pltpu.VMEM((_NCHT, _CH), jnp.int32),
        pltpu.VMEM((_CH, _D), jnp.float32),
        pltpu.VMEM((_CH, _D), jnp.float32),
        pltpu.VMEM_SHARED((_NPAD, _D), jnp.float32),
        pltpu.SemaphoreType.DMA,
        pltpu.SemaphoreType.DMA,
        pltpu.SemaphoreType.DMA,
        pltpu.SemaphoreType.DMA,
    ],
)
def _agg_sc(y_hbm, src_hbm, dst_hbm, zero_hbm, out_hbm,
            sidx, didx, r0, r1, acc, sg0, sg1, ss0, ss1):
    _agg_body(y_hbm, src_hbm, dst_hbm, zero_hbm, out_hbm,
              sidx, didx, r0, r1, acc, sg0, sg1, ss0, ss1)


_BR = 5120


def _mm1_body(x_ref, w_ref, d0_ref, d1_ref, y_ref, dinv_ref):
    deg = d0_ref[...] + d1_ref[...]
    dinv = lax.rsqrt(jnp.maximum(deg, 1e-12))
    dinv_ref[...] = dinv
    xw = jnp.dot(x_ref[...], w_ref[...], preferred_element_type=jnp.float32)
    y_ref[...] = xw * dinv


def _tc_mm1(xpad, W1, deg0, deg1):
    return pl.pallas_call(
        _mm1_body,
        grid=(_NPAD // _BR,),
        in_specs=[
            pl.BlockSpec((_BR, _D), lambda i: (i, 0)),
            pl.BlockSpec((_D, _D), lambda i: (0, 0)),
            pl.BlockSpec((_BR, 1), lambda i: (i, 0)),
            pl.BlockSpec((_BR, 1), lambda i: (i, 0)),
        ],
        out_specs=[
            pl.BlockSpec((_BR, _D), lambda i: (i, 0)),
            pl.BlockSpec((_BR, 1), lambda i: (i, 0)),
        ],
        out_shape=[
            jax.ShapeDtypeStruct((_NPAD, _D), jnp.float32),
            jax.ShapeDtypeStruct((_NPAD, 1), jnp.float32),
        ],
    )(xpad, W1, deg0, deg1)


def _mid_body(a0_ref, a1_ref, dinv_ref, b1_ref, w2_ref, y2_ref):
    dinv = dinv_ref[...]
    h1 = jnp.maximum((a0_ref[...] + a1_ref[...]) * dinv + b1_ref[...], 0.0)
    y2_ref[...] = jnp.dot(h1, w2_ref[...],
                          preferred_element_type=jnp.float32) * dinv


def _tc_mid(a0, a1, dinv, b1, W2):
    return pl.pallas_call(
        _mid_body,
        grid=(_NPAD // _BR,),
        in_specs=[
            pl.BlockSpec((_BR, _D), lambda i: (i, 0)),
            pl.BlockSpec((_BR, _D), lambda i: (i, 0)),
            pl.BlockSpec((_BR, 1), lambda i: (i, 0)),
            pl.BlockSpec((1, _D), lambda i: (0, 0)),
            pl.BlockSpec((_D, _D), lambda i: (0, 0)),
        ],
        out_specs=pl.BlockSpec((_BR, _D), lambda i: (i, 0)),
        out_shape=jax.ShapeDtypeStruct((_NPAD, _D), jnp.float32),
    )(a0, a1, dinv, b1, W2)


def _fin_body(a0_ref, a1_ref, dinv_ref, b2_ref, p1w_ref, p1b_ref,
              p2w_ref, p2b_ref, z_ref, h_ref):
    z = (a0_ref[...] + a1_ref[...]) * dinv_ref[...] + b2_ref[...]
    z_ref[...] = z
    t = jnp.maximum(
        jnp.dot(z, p1w_ref[...], preferred_element_type=jnp.float32)
        + p1b_ref[...], 0.0)
    h_ref[...] = jnp.dot(t, p2w_ref[...],
                         preferred_element_type=jnp.float32) + p2b_ref[...]


def _tc_fin(a0, a1, dinv, b2, P1w, P1b, P2w, P2b):
    return pl.pallas_call(
        _fin_body,
        grid=(_NPAD // _BR,),
        in_specs=[
            pl.BlockSpec((_BR, _D), lambda i: (i, 0)),
            pl.BlockSpec((_BR, _D), lambda i: (i, 0)),
            pl.BlockSpec((_BR, 1), lambda i: (i, 0)),
            pl.BlockSpec((1, _D), lambda i: (0, 0)),
            pl.BlockSpec((_D, _D), lambda i: (0, 0)),
            pl.BlockSpec((1, _D), lambda i: (0, 0)),
            pl.BlockSpec((_D, _DP), lambda i: (0, 0)),
            pl.BlockSpec((1, _DP), lambda i: (0, 0)),
        ],
        out_specs=[
            pl.BlockSpec((_BR, _D), lambda i: (i, 0)),
            pl.BlockSpec((_BR, _DP), lambda i: (i, 0)),
        ],
        out_shape=[
            jax.ShapeDtypeStruct((_NPAD, _D), jnp.float32),
            jax.ShapeDtypeStruct((_NPAD, _DP), jnp.float32),
        ],
    )(a0, a1, dinv, b2, P1w, P1b, P2w, P2b)


def kernel(x, edge_index, W1, b1, W2, b2, P1w, P1b, P2w, P2b):
    src = edge_index[0]
    dst = edge_index[1]
    loop = jnp.arange(_N, dtype=jnp.int32)
    npadd = _EPAD - _E - _N
    pad_src = (jnp.arange(npadd, dtype=jnp.int32) * 7) % _N
    pad_dst = _N + jnp.arange(npadd, dtype=jnp.int32) % (_NPAD - _N)
    srcp = jnp.concatenate([src, loop, pad_src])
    dstp = jnp.concatenate([dst, loop, pad_dst]).reshape(32, _NCHT, _CH)

    xpad = jnp.zeros((_NPAD, _D), jnp.float32).at[:_N].set(x)
    zeros1 = jnp.zeros((_NPAD,), jnp.float32)
    zeros2 = jnp.zeros((_NPAD, _D), jnp.float32)

    deg = _deg_sc(dstp, zeros1)
    deg0 = deg[:_NPAD].reshape(_NPAD, 1)
    deg1 = deg[_NPAD:].reshape(_NPAD, 1)

    y1, dinv = _tc_mm1(xpad, W1, deg0, deg1)

    agg1 = _agg_sc(y1, srcp, dstp, zeros2)
    y2 = _tc_mid(agg1[:_NPAD], agg1[_NPAD:], dinv, b1.reshape(1, _D), W2)

    agg2 = _agg_sc(y2, srcp, dstp, zeros2)
    z, h = _tc_fin(agg2[:_NPAD], agg2[_NPAD:], dinv, b2.reshape(1, _D),
                   P1w, P1b.reshape(1, _D), P2w, P2b.reshape(1, _DP))
    return (z[:_N], h[:_N])


# speedup vs baseline: 27.3872x; 1.0508x over previous
import functools

import jax
import jax.numpy as jnp
import numpy as np
from jax import lax
from jax.experimental import pallas as pl
from jax.experimental.pallas import tpu as pltpu
from jax.experimental.pallas import tpu_sc as plsc

_N = 10000
_E = 320000
_D = 128
_DP = 64
_NPAD = 10240
_EPAD = 331776
_EPW = _EPAD // 32
_CH = 64
_NCHT = _EPW // _CH
_RPT = _NPAD // 16

_NC, _NS = 2, 16

_NDUM = _EPAD - _E - _N
_TAIL_SRC = jnp.asarray(np.concatenate([
    np.arange(_N, dtype=np.int32),
    (np.arange(_NDUM, dtype=np.int32) * 7) % _N,
]))
_TAIL_DST = jnp.asarray(np.concatenate([
    np.arange(_N, dtype=np.int32),
    _N + np.arange(_NDUM, dtype=np.int32) % (_NPAD - _N),
]))


def _mesh():
    return plsc.VectorSubcoreMesh(core_axis_name="c", subcore_axis_name="s")


def _deg_body(dst_hbm, zero_hbm, out_hbm, didx, ones_v, stg, dacc, sem):
    c = lax.axis_index("c")
    s = lax.axis_index("s")
    wid = c * _NS + s
    pltpu.sync_copy(zero_hbm.at[pl.ds(s * _RPT, _RPT)], stg)
    pltpu.sync_copy(stg, dacc.at[pl.ds(s * _RPT, _RPT)])
    pltpu.sync_copy(dst_hbm.at[wid], didx)
    for i in range(_CH // 16):
        ones_v[pl.ds(i * 16, 16)] = jnp.ones((16,), jnp.float32)
    plsc.subcore_barrier()

    def body(j, carry):
        pltpu.sync_copy(ones_v, dacc.at[didx.at[j]], add=True)
        return carry

    lax.fori_loop(0, _NCHT, body, 0)
    plsc.subcore_barrier()
    pltpu.sync_copy(dacc.at[pl.ds(s * _RPT, _RPT)], stg)
    pltpu.sync_copy(stg, out_hbm.at[pl.ds(c * _NPAD + s * _RPT, _RPT)])


@functools.partial(
    pl.kernel,
    mesh=_mesh(),
    out_type=jax.ShapeDtypeStruct((_NC * _NPAD,), jnp.float32),
    scratch_types=[
        pltpu.VMEM((_NCHT, _CH), jnp.int32),
        pltpu.VMEM((_CH,), jnp.float32),
        pltpu.VMEM((_RPT,), jnp.float32),
        pltpu.VMEM_SHARED((_NPAD,), jnp.float32),
        pltpu.SemaphoreType.DMA,
    ],
)
def _deg_sc(dst_hbm, zero_hbm, out_hbm, didx, ones_v, stg, dacc, sem):
    _deg_body(dst_hbm, zero_hbm, out_hbm, didx, ones_v, stg, dacc, sem)


_WBR = 72


def _agg_body(y_hbm, src_hbm, dst_hbm, zero_hbm, out_hbm,
              sidx, didx, r0, r1, acc,
              sg0, sg1, ss0, ss1):
    c = lax.axis_index("c")
    s = lax.axis_index("s")
    wid = c * _NS + s
    pltpu.sync_copy(zero_hbm.at[pl.ds(s * _RPT, _CH)], r0)
    for k in range(_RPT // _CH):
        pltpu.sync_copy(r0, acc.at[pl.ds(s * _RPT + k * _CH, _CH)])
    pltpu.sync_copy(src_hbm.at[pl.ds(wid * _EPW, _EPW)], sidx)
    pltpu.sync_copy(dst_hbm.at[wid], didx)
    plsc.subcore_barrier()

    def sl(j):
        return sidx.at[pl.ds(j * _CH, _CH)]

    def wait_g(buf, sem, j):
        pltpu.make_async_copy(y_hbm.at[sl(j)], buf, sem).wait()

    def wait_s(buf, sem, j):
        pltpu.make_async_copy(buf, acc.at[didx.at[j]], sem).wait()

    pltpu.async_copy(y_hbm.at[sl(0)], r0, sg0)

    def body(jj, carry):
        j = 2 * jj
        @pl.when(jj > 0)
        def _():
            wait_s(r1, ss1, j - 1)
        pltpu.async_copy(y_hbm.at[sl(j + 1)], r1, sg1)
        wait_g(r0, sg0, j)
        pltpu.async_copy(r0, acc.at[didx.at[j]], ss0, add=True)
        wait_s(r0, ss0, j)
        @pl.when(j + 2 < _NCHT)
        def _():
            pltpu.async_copy(y_hbm.at[sl(j + 2)], r0, sg0)
        wait_g(r1, sg1, j + 1)
        pltpu.async_copy(r1, acc.at[didx.at[j + 1]], ss1, add=True)
        return carry

    lax.fori_loop(0, _NCHT // 2, body, 0)
    last = _NCHT - 1
    if _NCHT % 2:
        wait_s(r1, ss1, last - 1)
        wait_g(r0, sg0, last)
        pltpu.async_copy(r0, acc.at[didx.at[last]], ss0, add=True)
        wait_s(r0, ss0, last)
    else:
        wait_s(r1, ss1, last)
    plsc.subcore_barrier()
    for k in range(_RPT // _CH):
        pltpu.sync_copy(acc.at[pl.ds(s * _RPT + k * _CH, _CH)], r0)
        pltpu.sync_copy(
            r0, out_hbm.at[pl.ds(c * _NPAD + s * _RPT + k * _CH, _CH)])


@functools.partial(
    pl.kernel,
    mesh=_mesh(),
    out_type=jax.ShapeDtypeStruct((_NC * _NPAD, _D), jnp.float32),
    scratch_types=[
        pltpu.VMEM((_EPW,), jnp.int32),
        pltpu.VMEM((_NCHT, _CH), jnp.int32),
        pltpu.VMEM((_CH, _D), jnp.float32),
        pltpu.VMEM((_CH, _D), jnp.float32),
        pltpu.VMEM_SHARED((_NPAD, _D), jnp.float32),
        pltpu.SemaphoreType.DMA,
        pltpu.SemaphoreType.DMA,
        pltpu.SemaphoreType.DMA,
        pltpu.SemaphoreType.DMA,
    ],
)
def _agg_sc(y_hbm, src_hbm, dst_hbm, zero_hbm, out_hbm,
            sidx, didx, r0, r1, acc, sg0, sg1, ss0, ss1):
    _agg_body(y_hbm, src_hbm, dst_hbm, zero_hbm, out_hbm,
              sidx, didx, r0, r1, acc, sg0, sg1, ss0, ss1)


_BR = 5120


def _mm1_body(x_ref, w_ref, d_ref, y_ref, dinv_ref):
    deg = d_ref[0] + d_ref[1]
    dinv = lax.rsqrt(jnp.maximum(deg, 1e-12))
    dinv_ref[...] = dinv
    xw = jnp.dot(x_ref[...], w_ref[...], preferred_element_type=jnp.float32)
    y_ref[...] = xw * dinv


def _tc_mm1(xpad, W1, deg):
    return pl.pallas_call(
        _mm1_body,
        grid=(_NPAD // _BR,),
        in_specs=[
            pl.BlockSpec((_BR, _D), lambda i: (i, 0)),
            pl.BlockSpec((_D, _D), lambda i: (0, 0)),
            pl.BlockSpec((2, _BR, 1), lambda i: (0, i, 0)),
        ],
        out_specs=[
            pl.BlockSpec((_BR, _D), lambda i: (i, 0)),
            pl.BlockSpec((_BR, 1), lambda i: (i, 0)),
        ],
        out_shape=[
            jax.ShapeDtypeStruct((_NPAD, _D), jnp.float32),
            jax.ShapeDtypeStruct((_NPAD, 1), jnp.float32),
        ],
    )(xpad, W1, deg)


def _mid_body(a_ref, dinv_ref, b1_ref, w2_ref, y2_ref):
    dinv = dinv_ref[...]
    h1 = jnp.maximum((a_ref[0] + a_ref[1]) * dinv + b1_ref[...], 0.0)
    y2_ref[...] = jnp.dot(h1, w2_ref[...],
                          preferred_element_type=jnp.float32) * dinv


def _tc_mid(a, dinv, b1, W2):
    return pl.pallas_call(
        _mid_body,
        grid=(_NPAD // _BR,),
        in_specs=[
            pl.BlockSpec((2, _BR, _D), lambda i: (0, i, 0)),
            pl.BlockSpec((_BR, 1), lambda i: (i, 0)),
            pl.BlockSpec((1, _D), lambda i: (0, 0)),
            pl.BlockSpec((_D, _D), lambda i: (0, 0)),
        ],
        out_specs=pl.BlockSpec((_BR, _D), lambda i: (i, 0)),
        out_shape=jax.ShapeDtypeStruct((_NPAD, _D), jnp.float32),
    )(a, dinv, b1, W2)


def _fin_body(a_ref, dinv_ref, b2_ref, p1w_ref, p1b_ref,
              p2w_ref, p2b_ref, z_ref, h_ref):
    z = (a_ref[0] + a_ref[1]) * dinv_ref[...] + b2_ref[...]
    z_ref[...] = z
    t = jnp.maximum(
        jnp.dot(z, p1w_ref[...], preferred_element_type=jnp.float32)
        + p1b_ref[...], 0.0)
    h_ref[...] = jnp.dot(t, p2w_ref[...],
                         preferred_element_type=jnp.float32) + p2b_ref[...]


def _tc_fin(a, dinv, b2, P1w, P1b, P2w, P2b):
    return pl.pallas_call(
        _fin_body,
        grid=(_NPAD // _BR,),
        in_specs=[
            pl.BlockSpec((2, _BR, _D), lambda i: (0, i, 0)),
            pl.BlockSpec((_BR, 1), lambda i: (i, 0)),
            pl.BlockSpec((1, _D), lambda i: (0, 0)),
            pl.BlockSpec((_D, _D), lambda i: (0, 0)),
            pl.BlockSpec((1, _D), lambda i: (0, 0)),
            pl.BlockSpec((_D, _DP), lambda i: (0, 0)),
            pl.BlockSpec((1, _DP), lambda i: (0, 0)),
        ],
        out_specs=[
            pl.BlockSpec((_BR, _D), lambda i: (i, 0)),
            pl.BlockSpec((_BR, _DP), lambda i: (i, 0)),
        ],
        out_shape=[
            jax.ShapeDtypeStruct((_N, _D), jnp.float32),
            jax.ShapeDtypeStruct((_N, _DP), jnp.float32),
        ],
    )(a, dinv, b2, P1w, P1b, P2w, P2b)


def kernel(x, edge_index, W1, b1, W2, b2, P1w, P1b, P2w, P2b):
    src = edge_index[0]
    dst = edge_index[1]
    dstp = jnp.concatenate([dst, _TAIL_DST]).reshape(32, _NCHT, _CH)
    srcp = jnp.concatenate([src, _TAIL_SRC])

    xpad = jnp.zeros((_NPAD, _D), jnp.float32).at[:_N].set(x)
    zeros1 = jnp.zeros((_NPAD,), jnp.float32)
    zeros2 = jnp.zeros((_NPAD, _D), jnp.float32)

    deg = _deg_sc(dstp, zeros1).reshape(_NC, _NPAD, 1)

    y1, dinv = _tc_mm1(xpad, W1, deg)

    agg1 = _agg_sc(y1, srcp, dstp, zeros2).reshape(_NC, _NPAD, _D)
    y2 = _tc_mid(agg1, dinv, b1.reshape(1, _D), W2)

    agg2 = _agg_sc(y2, srcp, dstp, zeros2).reshape(_NC, _NPAD, _D)
    z, h = _tc_fin(agg2, dinv, b2.reshape(1, _D),
                   P1w, P1b.reshape(1, _D), P2w, P2b.reshape(1, _DP))
    return (z, h)


# speedup vs baseline: 28.2312x; 1.0308x over previous
import functools

import jax
import jax.numpy as jnp
import numpy as np
from jax import lax
from jax.experimental import pallas as pl
from jax.experimental.pallas import tpu as pltpu
from jax.experimental.pallas import tpu_sc as plsc

_N = 10000
_E = 320000
_D = 128
_DP = 64
_NPAD = 10240
_EPAD = 331776
_EPW = _EPAD // 32
_CH = 64
_NCHT = _EPW // _CH
_RPT = _NPAD // 16

_NC, _NS = 2, 16

_NDUM = _EPAD - _E - _N
_TAIL_SRC = jnp.asarray(np.concatenate([
    np.arange(_N, dtype=np.int32),
    (np.arange(_NDUM, dtype=np.int32) * 7) % _N,
]))
_TAIL_DST = jnp.asarray(np.concatenate([
    np.arange(_N, dtype=np.int32),
    _N + np.arange(_NDUM, dtype=np.int32) % (_NPAD - _N),
]))


def _mesh():
    return plsc.VectorSubcoreMesh(core_axis_name="c", subcore_axis_name="s")


def _deg_body(dst_hbm, zero_hbm, out_hbm, didx, ones_v, stg, dacc, sem):
    c = lax.axis_index("c")
    s = lax.axis_index("s")
    wid = c * _NS + s
    pltpu.sync_copy(zero_hbm.at[pl.ds(s * _RPT, _RPT)], stg)
    pltpu.sync_copy(stg, dacc.at[pl.ds(s * _RPT, _RPT)])
    pltpu.sync_copy(dst_hbm.at[wid], didx)
    for i in range(_CH // 16):
        ones_v[pl.ds(i * 16, 16)] = jnp.ones((16,), jnp.float32)
    plsc.subcore_barrier()

    def body(j, carry):
        pltpu.sync_copy(ones_v, dacc.at[didx.at[j]], add=True)
        return carry

    lax.fori_loop(0, _NCHT, body, 0)
    plsc.subcore_barrier()
    pltpu.sync_copy(dacc.at[pl.ds(s * _RPT, _RPT)], stg)
    pltpu.sync_copy(stg, out_hbm.at[pl.ds(c * _NPAD + s * _RPT, _RPT)])


@functools.partial(
    pl.kernel,
    mesh=_mesh(),
    out_type=jax.ShapeDtypeStruct((_NC * _NPAD,), jnp.float32),
    scratch_types=[
        pltpu.VMEM((_NCHT, _CH), jnp.int32),
        pltpu.VMEM((_CH,), jnp.float32),
        pltpu.VMEM((_RPT,), jnp.float32),
        pltpu.VMEM_SHARED((_NPAD,), jnp.float32),
        pltpu.SemaphoreType.DMA,
    ],
)
def _deg_sc(dst_hbm, zero_hbm, out_hbm, didx, ones_v, stg, dacc, sem):
    _deg_body(dst_hbm, zero_hbm, out_hbm, didx, ones_v, stg, dacc, sem)


_WBR = 72


def _agg_body(y_hbm, src_hbm, dst_hbm, zero_hbm, out_hbm,
              sidx, didx, r0, r1, acc,
              sg0, sg1, ss0, ss1):
    c = lax.axis_index("c")
    s = lax.axis_index("s")
    wid = c * _NS + s
    pltpu.sync_copy(zero_hbm.at[pl.ds(s * _RPT, _CH)], r0)
    for k in range(_RPT // _CH):
        pltpu.sync_copy(r0, acc.at[pl.ds(s * _RPT + k * _CH, _CH)])
    pltpu.sync_copy(src_hbm.at[pl.ds(wid * _EPW, _EPW)], sidx)
    pltpu.sync_copy(dst_hbm.at[wid], didx)
    plsc.subcore_barrier()

    def sl(j):
        return sidx.at[pl.ds(j * _CH, _CH)]

    def wait_g(buf, sem, j):
        pltpu.make_async_copy(y_hbm.at[sl(j)], buf, sem).wait()

    def wait_s(buf, sem, j):
        pltpu.make_async_copy(buf, acc.at[didx.at[j]], sem).wait()

    pltpu.async_copy(y_hbm.at[sl(0)], r0, sg0)

    def body(jj, carry):
        j = 2 * jj
        @pl.when(jj > 0)
        def _():
            wait_s(r1, ss1, j - 1)
        pltpu.async_copy(y_hbm.at[sl(j + 1)], r1, sg1)
        wait_g(r0, sg0, j)
        pltpu.async_copy(r0, acc.at[didx.at[j]], ss0, add=True)
        wait_s(r0, ss0, j)
        @pl.when(j + 2 < _NCHT)
        def _():
            pltpu.async_copy(y_hbm.at[sl(j + 2)], r0, sg0)
        wait_g(r1, sg1, j + 1)
        pltpu.async_copy(r1, acc.at[didx.at[j + 1]], ss1, add=True)
        return carry

    lax.fori_loop(0, _NCHT // 2, body, 0)
    last = _NCHT - 1
    if _NCHT % 2:
        wait_s(r1, ss1, last - 1)
        wait_g(r0, sg0, last)
        pltpu.async_copy(r0, acc.at[didx.at[last]], ss0, add=True)
        wait_s(r0, ss0, last)
    else:
        wait_s(r1, ss1, last)
    plsc.subcore_barrier()
    for k in range(_RPT // _CH):
        pltpu.sync_copy(acc.at[pl.ds(s * _RPT + k * _CH, _CH)], r0)
        pltpu.sync_copy(
            r0, out_hbm.at[pl.ds(c * _NPAD + s * _RPT + k * _CH, _CH)])


@functools.partial(
    pl.kernel,
    mesh=_mesh(),
    out_type=jax.ShapeDtypeStruct((_NC * _NPAD, _D), jnp.float32),
    scratch_types=[
        pltpu.VMEM((_EPW,), jnp.int32),
        pltpu.VMEM((_NCHT, _CH), jnp.int32),
        pltpu.VMEM((_CH, _D), jnp.float32),
        pltpu.VMEM((_CH, _D), jnp.float32),
        pltpu.VMEM_SHARED((_NPAD, _D), jnp.float32),
        pltpu.SemaphoreType.DMA,
        pltpu.SemaphoreType.DMA,
        pltpu.SemaphoreType.DMA,
        pltpu.SemaphoreType.DMA,
    ],
)
def _agg_sc(y_hbm, src_hbm, dst_hbm, zero_hbm, out_hbm,
            sidx, didx, r0, r1, acc, sg0, sg1, ss0, ss1):
    _agg_body(y_hbm, src_hbm, dst_hbm, zero_hbm, out_hbm,
              sidx, didx, r0, r1, acc, sg0, sg1, ss0, ss1)


_BR = 5120


def _xw_body(x_ref, w_ref, xw_ref):
    xw_ref[...] = jnp.dot(x_ref[...], w_ref[...],
                          preferred_element_type=jnp.float32)


def _tc_xw(xpad, W1):
    return pl.pallas_call(
        _xw_body,
        grid=(_NPAD // _BR,),
        in_specs=[
            pl.BlockSpec((_BR, _D), lambda i: (i, 0)),
            pl.BlockSpec((_D, _D), lambda i: (0, 0)),
        ],
        out_specs=pl.BlockSpec((_BR, _D), lambda i: (i, 0)),
        out_shape=jax.ShapeDtypeStruct((_NPAD, _D), jnp.float32),
    )(xpad, W1)


def _scale_body(xw_ref, d_ref, y_ref, dinv_ref):
    dinv = lax.rsqrt(jnp.maximum(d_ref[...], 1e-12))
    dinv_ref[...] = dinv
    y_ref[...] = xw_ref[...] * dinv


def _tc_scale(xw, deg):
    return pl.pallas_call(
        _scale_body,
        grid=(_NPAD // _BR,),
        in_specs=[
            pl.BlockSpec((_BR, _D), lambda i: (i, 0)),
            pl.BlockSpec((_BR, 1), lambda i: (i, 0)),
        ],
        out_specs=[
            pl.BlockSpec((_BR, _D), lambda i: (i, 0)),
            pl.BlockSpec((_BR, 1), lambda i: (i, 0)),
        ],
        out_shape=[
            jax.ShapeDtypeStruct((_NPAD, _D), jnp.float32),
            jax.ShapeDtypeStruct((_NPAD, 1), jnp.float32),
        ],
    )(xw, deg)


def _mid_body(a_ref, dinv_ref, b1_ref, w2_ref, y2_ref):
    dinv = dinv_ref[...]
    h1 = jnp.maximum((a_ref[0] + a_ref[1]) * dinv + b1_ref[...], 0.0)
    y2_ref[...] = jnp.dot(h1, w2_ref[...],
                          preferred_element_type=jnp.float32) * dinv


def _tc_mid(a, dinv, b1, W2):
    return pl.pallas_call(
        _mid_body,
        grid=(_NPAD // _BR,),
        in_specs=[
            pl.BlockSpec((2, _BR, _D), lambda i: (0, i, 0)),
            pl.BlockSpec((_BR, 1), lambda i: (i, 0)),
            pl.BlockSpec((1, _D), lambda i: (0, 0)),
            pl.BlockSpec((_D, _D), lambda i: (0, 0)),
        ],
        out_specs=pl.BlockSpec((_BR, _D), lambda i: (i, 0)),
        out_shape=jax.ShapeDtypeStruct((_NPAD, _D), jnp.float32),
    )(a, dinv, b1, W2)


def _fin_body(a_ref, dinv_ref, b2_ref, p1w_ref, p1b_ref,
              p2w_ref, p2b_ref, z_ref, h_ref):
    dinv = dinv_ref[...]
    z = (a_ref[0] + a_ref[1]) * dinv + b2_ref[...]
    z_ref[...] = z
    t = jnp.maximum(
        jnp.dot(z, p1w_ref[...], preferred_element_type=jnp.float32)
        + p1b_ref[...], 0.0)
    h_ref[...] = jnp.dot(t, p2w_ref[...],
                         preferred_element_type=jnp.float32) + p2b_ref[...]


def _tc_fin(a, dinv, b2, P1w, P1b, P2w, P2b):
    return pl.pallas_call(
        _fin_body,
        grid=(_NPAD // _BR,),
        in_specs=[
            pl.BlockSpec((2, _BR, _D), lambda i: (0, i, 0)),
            pl.BlockSpec((_BR, 1), lambda i: (i, 0)),
            pl.BlockSpec((1, _D), lambda i: (0, 0)),
            pl.BlockSpec((_D, _D), lambda i: (0, 0)),
            pl.BlockSpec((1, _D), lambda i: (0, 0)),
            pl.BlockSpec((_D, _DP), lambda i: (0, 0)),
            pl.BlockSpec((1, _DP), lambda i: (0, 0)),
        ],
        out_specs=[
            pl.BlockSpec((_BR, _D), lambda i: (i, 0)),
            pl.BlockSpec((_BR, _DP), lambda i: (i, 0)),
        ],
        out_shape=[
            jax.ShapeDtypeStruct((_N, _D), jnp.float32),
            jax.ShapeDtypeStruct((_N, _DP), jnp.float32),
        ],
    )(a, dinv, b2, P1w, P1b, P2w, P2b)


def kernel(x, edge_index, W1, b1, W2, b2, P1w, P1b, P2w, P2b):
    src = edge_index[0]
    dst = edge_index[1]
    dstp = jnp.concatenate([dst, _TAIL_DST]).reshape(32, _NCHT, _CH)
    srcp = jnp.concatenate([src, _TAIL_SRC])

    xpad = jnp.zeros((_NPAD, _D), jnp.float32).at[:_N].set(x)
    zeros1 = jnp.zeros((_NPAD,), jnp.float32)
    zeros2 = jnp.zeros((_NPAD, _D), jnp.float32)

    degp = _deg_sc(dstp, zeros1)
    deg = (degp[:_NPAD] + degp[_NPAD:]).reshape(_NPAD, 1)
    xw1 = _tc_xw(xpad, W1)
    y1, dinv = _tc_scale(xw1, deg)

    agg1 = _agg_sc(y1, srcp, dstp, zeros2).reshape(_NC, _NPAD, _D)
    y2 = _tc_mid(agg1, dinv, b1.reshape(1, _D), W2)

    agg2 = _agg_sc(y2, srcp, dstp, zeros2).reshape(_NC, _NPAD, _D)
    z, h = _tc_fin(agg2, dinv, b2.reshape(1, _D),
                   P1w, P1b.reshape(1, _D), P2w, P2b.reshape(1, _DP))
    return (z, h)


# speedup vs baseline: 32.6355x; 1.1560x over previous
import functools

import jax
import jax.numpy as jnp
import numpy as np
from jax import lax
from jax.experimental import pallas as pl
from jax.experimental.pallas import tpu as pltpu
from jax.experimental.pallas import tpu_sc as plsc

_N = 10000
_E = 320000
_D = 128
_DP = 64
_NPAD = 10112
_EPAD = 331776
_EPW = _EPAD // 32
_CH = 128
_NCHT = _EPW // _CH
_NCHA = 41
_RPT = _NPAD // 16

_NC, _NS = 2, 16

_NDUM = _EPAD - _E - _N
_TAIL_SRC = jnp.asarray(np.concatenate([
    np.arange(_N, dtype=np.int32),
    (np.arange(_NDUM, dtype=np.int32) * 7) % _N,
]))
_TAIL_DST = jnp.asarray(np.concatenate([
    np.arange(_N, dtype=np.int32),
    _N + np.arange(_NDUM, dtype=np.int32) % (_NPAD - _N),
]))


def _mesh():
    return plsc.VectorSubcoreMesh(core_axis_name="c", subcore_axis_name="s")


def _deg_body(dst_hbm, zero_hbm, out_hbm, didx, ones_v, stg, dacc, sem):
    c = lax.axis_index("c")
    s = lax.axis_index("s")
    wid = c * _NS + s
    pltpu.sync_copy(zero_hbm.at[pl.ds(s * _RPT, _RPT)], stg)
    pltpu.sync_copy(stg, dacc.at[pl.ds(s * _RPT, _RPT)])
    pltpu.sync_copy(dst_hbm.at[wid], didx)
    for i in range(_CH // 16):
        ones_v[pl.ds(i * 16, 16)] = jnp.ones((16,), jnp.float32)
    plsc.subcore_barrier()

    def body(j, carry):
        pltpu.sync_copy(ones_v, dacc.at[didx.at[j]], add=True)
        return carry

    lax.fori_loop(0, _NCHT, body, 0)
    plsc.subcore_barrier()
    pltpu.sync_copy(dacc.at[pl.ds(s * _RPT, _RPT)], stg)
    pltpu.sync_copy(stg, out_hbm.at[pl.ds(c * _NPAD + s * _RPT, _RPT)])


@functools.partial(
    pl.kernel,
    mesh=_mesh(),
    out_type=jax.ShapeDtypeStruct((_NC * _NPAD,), jnp.float32),
    scratch_types=[
        pltpu.VMEM((_NCHT, _CH), jnp.int32),
        pltpu.VMEM((_CH,), jnp.float32),
        pltpu.VMEM((_RPT,), jnp.float32),
        pltpu.VMEM_SHARED((_NPAD,), jnp.float32),
        pltpu.SemaphoreType.DMA,
    ],
)
def _deg_sc(dst_hbm, zero_hbm, out_hbm, didx, ones_v, stg, dacc, sem):
    _deg_body(dst_hbm, zero_hbm, out_hbm, didx, ones_v, stg, dacc, sem)


def _agg_body(y_hbm, src_hbm, dst_hbm, zero_hbm, out_hbm,
              sidx, didx, r0, r1, acc,
              sg0, sg1, ss0, ss1):
    c = lax.axis_index("c")
    s = lax.axis_index("s")
    wid = c * _NS + s
    pltpu.sync_copy(zero_hbm.at[pl.ds(s * _RPT, _CH)], r0)
    for k in range(_RPT // _CH):
        pltpu.sync_copy(r0, acc.at[pl.ds(s * _RPT + k * _CH, _CH)])
    rem = _RPT - (_RPT // _CH) * _CH
    if rem:
        pltpu.sync_copy(r0.at[pl.ds(0, rem)],
                        acc.at[pl.ds(s * _RPT + (_RPT // _CH) * _CH, rem)])
    pltpu.sync_copy(src_hbm.at[pl.ds(wid * _EPW, _NCHA * _CH)], sidx)
    pltpu.sync_copy(dst_hbm.at[wid], didx)
    plsc.subcore_barrier()

    def run(j0, n):
        def sl(r):
            return sidx.at[pl.ds(r * _CH, _CH)]

        def wait_g(buf, sem, r):
            pltpu.make_async_copy(y_hbm.at[sl(r)], buf, sem).wait()

        def wait_s(buf, sem, j):
            pltpu.make_async_copy(buf, acc.at[didx.at[j]], sem).wait()

        pltpu.async_copy(y_hbm.at[sl(0)], r0, sg0)

        def body(jj, carry):
            r = 2 * jj
            j = j0 + r
            @pl.when(jj > 0)
            def _():
                wait_s(r1, ss1, j - 1)
            pltpu.async_copy(y_hbm.at[sl(r + 1)], r1, sg1)
            wait_g(r0, sg0, r)
            pltpu.async_copy(r0, acc.at[didx.at[j]], ss0, add=True)
            wait_s(r0, ss0, j)
            @pl.when(r + 2 < n)
            def _():
                pltpu.async_copy(y_hbm.at[sl(r + 2)], r0, sg0)
            wait_g(r1, sg1, r + 1)
            pltpu.async_copy(r1, acc.at[didx.at[j + 1]], ss1, add=True)
            return carry

        lax.fori_loop(0, n // 2, body, 0)
        if n % 2:
            wait_s(r1, ss1, j0 + n - 2)
            wait_g(r0, sg0, n - 1)
            pltpu.async_copy(r0, acc.at[didx.at[j0 + n - 1]], ss0, add=True)
            wait_s(r0, ss0, j0 + n - 1)
        else:
            wait_s(r1, ss1, j0 + n - 1)

    run(0, _NCHA)
    pltpu.sync_copy(
        src_hbm.at[pl.ds(wid * _EPW + _NCHA * _CH, (_NCHT - _NCHA) * _CH)],
        sidx.at[pl.ds(0, (_NCHT - _NCHA) * _CH)])
    run(_NCHA, _NCHT - _NCHA)

    plsc.subcore_barrier()
    for k in range(_RPT // _CH):
        pltpu.sync_copy(acc.at[pl.ds(s * _RPT + k * _CH, _CH)], r0)
        pltpu.sync_copy(
            r0, out_hbm.at[pl.ds(c * _NPAD + s * _RPT + k * _CH, _CH)])
    if rem:
        base = (_RPT // _CH) * _CH
        pltpu.sync_copy(acc.at[pl.ds(s * _RPT + base, rem)], r0.at[pl.ds(0, rem)])
        pltpu.sync_copy(r0.at[pl.ds(0, rem)],
                        out_hbm.at[pl.ds(c * _NPAD + s * _RPT + base, rem)])


@functools.partial(
    pl.kernel,
    mesh=_mesh(),
    out_type=jax.ShapeDtypeStruct((_NC * _NPAD, _D), jnp.float32),
    scratch_types=[
        pltpu.VMEM((_NCHA * _CH,), jnp.int32),
        pltpu.VMEM((_NCHT, _CH), jnp.int32),
        pltpu.VMEM((_CH, _D), jnp.float32),
        pltpu.VMEM((_CH, _D), jnp.float32),
        pltpu.VMEM_SHARED((_NPAD, _D), jnp.float32),
        pltpu.SemaphoreType.DMA,
        pltpu.SemaphoreType.DMA,
        pltpu.SemaphoreType.DMA,
        pltpu.SemaphoreType.DMA,
    ],
)
def _agg_sc(y_hbm, src_hbm, dst_hbm, zero_hbm, out_hbm,
            sidx, didx, r0, r1, acc, sg0, sg1, ss0, ss1):
    _agg_body(y_hbm, src_hbm, dst_hbm, zero_hbm, out_hbm,
              sidx, didx, r0, r1, acc, sg0, sg1, ss0, ss1)


_BR = 5056


def _xw_body(x_ref, w_ref, xw_ref):
    xw_ref[...] = jnp.dot(x_ref[...], w_ref[...],
                          preferred_element_type=jnp.float32)


def _tc_xw(xpad, W1):
    return pl.pallas_call(
        _xw_body,
        grid=(_NPAD // _BR,),
        in_specs=[
            pl.BlockSpec((_BR, _D), lambda i: (i, 0)),
            pl.BlockSpec((_D, _D), lambda i: (0, 0)),
        ],
        out_specs=pl.BlockSpec((_BR, _D), lambda i: (i, 0)),
        out_shape=jax.ShapeDtypeStruct((_NPAD, _D), jnp.float32),
    )(xpad, W1)


def _scale_body(xw_ref, d_ref, y_ref, dinv_ref):
    dinv = lax.rsqrt(jnp.maximum(d_ref[...], 1e-12))
    dinv_ref[...] = dinv
    y_ref[...] = xw_ref[...] * dinv


def _tc_scale(xw, deg):
    return pl.pallas_call(
        _scale_body,
        grid=(_NPAD // _BR,),
        in_specs=[
            pl.BlockSpec((_BR, _D), lambda i: (i, 0)),
            pl.BlockSpec((_BR, 1), lambda i: (i, 0)),
        ],
        out_specs=[
            pl.BlockSpec((_BR, _D), lambda i: (i, 0)),
            pl.BlockSpec((_BR, 1), lambda i: (i, 0)),
        ],
        out_shape=[
            jax.ShapeDtypeStruct((_NPAD, _D), jnp.float32),
            jax.ShapeDtypeStruct((_NPAD, 1), jnp.float32),
        ],
    )(xw, deg)


def _mid_body(a_ref, dinv_ref, b1_ref, w2_ref, y2_ref):
    dinv = dinv_ref[...]
    h1 = jnp.maximum((a_ref[0] + a_ref[1]) * dinv + b1_ref[...], 0.0)
    y2_ref[...] = jnp.dot(h1, w2_ref[...],
                          preferred_element_type=jnp.float32) * dinv


def _tc_mid(a, dinv, b1, W2):
    return pl.pallas_call(
        _mid_body,
        grid=(_NPAD // _BR,),
        in_specs=[
            pl.BlockSpec((2, _BR, _D), lambda i: (0, i, 0)),
            pl.BlockSpec((_BR, 1), lambda i: (i, 0)),
            pl.BlockSpec((1, _D), lambda i: (0, 0)),
            pl.BlockSpec((_D, _D), lambda i: (0, 0)),
        ],
        out_specs=pl.BlockSpec((_BR, _D), lambda i: (i, 0)),
        out_shape=jax.ShapeDtypeStruct((_NPAD, _D), jnp.float32),
    )(a, dinv, b1, W2)


def _fin_body(a_ref, dinv_ref, b2_ref, p1w_ref, p1b_ref,
              p2w_ref, p2b_ref, z_ref, h_ref):
    dinv = dinv_ref[...]
    z = (a_ref[0] + a_ref[1]) * dinv + b2_ref[...]
    z_ref[...] = z
    t = jnp.maximum(
        jnp.dot(z, p1w_ref[...], preferred_element_type=jnp.float32)
        + p1b_ref[...], 0.0)
    h_ref[...] = jnp.dot(t, p2w_ref[...],
                         preferred_element_type=jnp.float32) + p2b_ref[...]


def _tc_fin(a, dinv, b2, P1w, P1b, P2w, P2b):
    return pl.pallas_call(
        _fin_body,
        grid=(_NPAD // _BR,),
        in_specs=[
            pl.BlockSpec((2, _BR, _D), lambda i: (0, i, 0)),
            pl.BlockSpec((_BR, 1), lambda i: (i, 0)),
            pl.BlockSpec((1, _D), lambda i: (0, 0)),
            pl.BlockSpec((_D, _D), lambda i: (0, 0)),
            pl.BlockSpec((1, _D), lambda i: (0, 0)),
            pl.BlockSpec((_D, _DP), lambda i: (0, 0)),
            pl.BlockSpec((1, _DP), lambda i: (0, 0)),
        ],
        out_specs=[
            pl.BlockSpec((_BR, _D), lambda i: (i, 0)),
            pl.BlockSpec((_BR, _DP), lambda i: (i, 0)),
        ],
        out_shape=[
            jax.ShapeDtypeStruct((_N, _D), jnp.float32),
            jax.ShapeDtypeStruct((_N, _DP), jnp.float32),
        ],
    )(a, dinv, b2, P1w, P1b, P2w, P2b)


def kernel(x, edge_index, W1, b1, W2, b2, P1w, P1b, P2w, P2b):
    src = edge_index[0]
    dst = edge_index[1]
    dstp = jnp.concatenate([dst, _TAIL_DST]).reshape(32, _NCHT, _CH)
    srcp = jnp.concatenate([src, _TAIL_SRC])

    xpad = jnp.zeros((_NPAD, _D), jnp.float32).at[:_N].set(x)
    zeros1 = jnp.zeros((_NPAD,), jnp.float32)
    zeros2 = jnp.zeros((_NPAD, _D), jnp.float32)

    degp = _deg_sc(dstp, zeros1)
    deg = (degp[:_NPAD] + degp[_NPAD:]).reshape(_NPAD, 1)
    xw1 = _tc_xw(xpad, W1)
    y1, dinv = _tc_scale(xw1, deg)

    agg1 = _agg_sc(y1, srcp, dstp, zeros2).reshape(_NC, _NPAD, _D)
    y2 = _tc_mid(agg1, dinv, b1.reshape(1, _D), W2)

    agg2 = _agg_sc(y2, srcp, dstp, zeros2).reshape(_NC, _NPAD, _D)
    z, h = _tc_fin(agg2, dinv, b2.reshape(1, _D),
                   P1w, P1b.reshape(1, _D), P2w, P2b.reshape(1, _DP))
    return (z, h)


# speedup vs baseline: 32.6392x; 1.0001x over previous
import functools

import jax
import jax.numpy as jnp
import numpy as np
from jax import lax
from jax.experimental import pallas as pl
from jax.experimental.pallas import tpu as pltpu
from jax.experimental.pallas import tpu_sc as plsc

_N = 10000
_E = 320000
_D = 128
_DP = 64
_NPAD = 10112
_EPAD = 331776
_EPW = _EPAD // 32
_CH = 128
_NCHT = _EPW // _CH
_NCHA = 40
_RPT = _NPAD // 16

_NC, _NS = 2, 16

_NDUM = _EPAD - _E - _N
_TAIL_SRC = np.concatenate([
    np.arange(_N, dtype=np.int32),
    (np.arange(_NDUM, dtype=np.int32) * 7) % _N,
])
_TAIL_DST = np.concatenate([
    np.arange(_N, dtype=np.int32),
    _N + np.arange(_NDUM, dtype=np.int32) % (_NPAD - _N),
])


def _mesh():
    return plsc.VectorSubcoreMesh(core_axis_name="c", subcore_axis_name="s")


def _deg_body(dst_hbm, zero_hbm, out_hbm, didx, ones_v, stg, dacc, sem):
    c = lax.axis_index("c")
    s = lax.axis_index("s")
    wid = c * _NS + s
    pltpu.sync_copy(zero_hbm.at[pl.ds(s * _RPT, _RPT)], stg)
    pltpu.sync_copy(stg, dacc.at[pl.ds(s * _RPT, _RPT)])
    pltpu.sync_copy(dst_hbm.at[wid], didx)
    for i in range(_CH // 16):
        ones_v[pl.ds(i * 16, 16)] = jnp.ones((16,), jnp.float32)
    plsc.subcore_barrier()

    def body(j, carry):
        pltpu.sync_copy(ones_v, dacc.at[didx.at[j]], add=True)
        return carry

    lax.fori_loop(0, _NCHT, body, 0)
    plsc.subcore_barrier()
    pltpu.sync_copy(dacc.at[pl.ds(s * _RPT, _RPT)], stg)
    pltpu.sync_copy(stg, out_hbm.at[pl.ds(c * _NPAD + s * _RPT, _RPT)])


@functools.partial(
    pl.kernel,
    mesh=_mesh(),
    out_type=jax.ShapeDtypeStruct((_NC * _NPAD,), jnp.float32),
    scratch_types=[
        pltpu.VMEM((_NCHT, _CH), jnp.int32),
        pltpu.VMEM((_CH,), jnp.float32),
        pltpu.VMEM((_RPT,), jnp.float32),
        pltpu.VMEM_SHARED((_NPAD,), jnp.float32),
        pltpu.SemaphoreType.DMA,
    ],
)
def _deg_sc(dst_hbm, zero_hbm, out_hbm, didx, ones_v, stg, dacc, sem):
    _deg_body(dst_hbm, zero_hbm, out_hbm, didx, ones_v, stg, dacc, sem)


def _agg_body(y_hbm, src_hbm, dst_hbm, zero_hbm, out_hbm,
              sidx, didx, r0, r1, acc,
              sg0, sg1, ss0, ss1):
    c = lax.axis_index("c")
    s = lax.axis_index("s")
    wid = c * _NS + s
    pltpu.sync_copy(zero_hbm.at[pl.ds(s * _RPT, _CH)], r0)
    for k in range(_RPT // _CH):
        pltpu.sync_copy(r0, acc.at[pl.ds(s * _RPT + k * _CH, _CH)])
    rem = _RPT - (_RPT // _CH) * _CH
    if rem:
        pltpu.sync_copy(r0.at[pl.ds(0, rem)],
                        acc.at[pl.ds(s * _RPT + (_RPT // _CH) * _CH, rem)])
    pltpu.sync_copy(src_hbm.at[wid, pl.ds(0, _NCHA)], sidx.at[pl.ds(0, _NCHA)])
    pltpu.sync_copy(dst_hbm.at[wid, pl.ds(0, _NCHA)], didx.at[pl.ds(0, _NCHA)])
    plsc.subcore_barrier()

    def run(n):
        def wait_g(buf, sem, r):
            pltpu.make_async_copy(y_hbm.at[sidx.at[r]], buf, sem).wait()

        def wait_s(buf, sem, r):
            pltpu.make_async_copy(buf, acc.at[didx.at[r]], sem).wait()

        pltpu.async_copy(y_hbm.at[sidx.at[0]], r0, sg0)

        def body(jj, carry):
            r = 2 * jj
            @pl.when(jj > 0)
            def _():
                wait_s(r1, ss1, r - 1)
            pltpu.async_copy(y_hbm.at[sidx.at[r + 1]], r1, sg1)
            wait_g(r0, sg0, r)
            pltpu.async_copy(r0, acc.at[didx.at[r]], ss0, add=True)
            wait_s(r0, ss0, r)
            @pl.when(r + 2 < n)
            def _():
                pltpu.async_copy(y_hbm.at[sidx.at[r + 2]], r0, sg0)
            wait_g(r1, sg1, r + 1)
            pltpu.async_copy(r1, acc.at[didx.at[r + 1]], ss1, add=True)
            return carry

        lax.fori_loop(0, n // 2, body, 0)
        if n % 2:
            wait_s(r1, ss1, n - 2)
            wait_g(r0, sg0, n - 1)
            pltpu.async_copy(r0, acc.at[didx.at[n - 1]], ss0, add=True)
            wait_s(r0, ss0, n - 1)
        else:
            wait_s(r1, ss1, n - 1)

    run(_NCHA)
    pltpu.sync_copy(src_hbm.at[wid, pl.ds(_NCHA, _NCHT - _NCHA)], sidx)
    pltpu.sync_copy(dst_hbm.at[wid, pl.ds(_NCHA, _NCHT - _NCHA)], didx)
    run(_NCHT - _NCHA)

    plsc.subcore_barrier()
    for k in range(_RPT // _CH):
        pltpu.sync_copy(acc.at[pl.ds(s * _RPT + k * _CH, _CH)], r0)
        pltpu.sync_copy(
            r0, out_hbm.at[pl.ds(c * _NPAD + s * _RPT + k * _CH, _CH)])
    if rem:
        base = (_RPT // _CH) * _CH
        pltpu.sync_copy(acc.at[pl.ds(s * _RPT + base, rem)], r0.at[pl.ds(0, rem)])
        pltpu.sync_copy(r0.at[pl.ds(0, rem)],
                        out_hbm.at[pl.ds(c * _NPAD + s * _RPT + base, rem)])


@functools.partial(
    pl.kernel,
    mesh=_mesh(),
    out_type=jax.ShapeDtypeStruct((_NC * _NPAD, _D), jnp.float32),
    scratch_types=[
        pltpu.VMEM((_NCHT - _NCHA, _CH), jnp.int32),
        pltpu.VMEM((_NCHT - _NCHA, _CH), jnp.int32),
        pltpu.VMEM((_CH, _D), jnp.float32),
        pltpu.VMEM((_CH, _D), jnp.float32),
        pltpu.VMEM_SHARED((_NPAD, _D), jnp.float32),
        pltpu.SemaphoreType.DMA,
        pltpu.SemaphoreType.DMA,
        pltpu.SemaphoreType.DMA,
        pltpu.SemaphoreType.DMA,
    ],
)
def _agg_sc(y_hbm, src_hbm, dst_hbm, zero_hbm, out_hbm,
            sidx, didx, r0, r1, acc, sg0, sg1, ss0, ss1):
    _agg_body(y_hbm, src_hbm, dst_hbm, zero_hbm, out_hbm,
              sidx, didx, r0, r1, acc, sg0, sg1, ss0, ss1)


_BR = 5056


def _xw_body(x_ref, w_ref, xw_ref):
    xw_ref[...] = jnp.dot(x_ref[...], w_ref[...],
                          preferred_element_type=jnp.float32)


def _tc_xw(xpad, W1):
    return pl.pallas_call(
        _xw_body,
        grid=(_NPAD // _BR,),
        in_specs=[
            pl.BlockSpec((_BR, _D), lambda i: (i, 0)),
            pl.BlockSpec((_D, _D), lambda i: (0, 0)),
        ],
        out_specs=pl.BlockSpec((_BR, _D), lambda i: (i, 0)),
        out_shape=jax.ShapeDtypeStruct((_NPAD, _D), jnp.float32),
    )(xpad, W1)


def _scale_body(xw_ref, d_ref, y_ref, dinv_ref):
    dinv = lax.rsqrt(jnp.maximum(d_ref[...], 1e-12))
    dinv_ref[...] = dinv
    y_ref[...] = xw_ref[...] * dinv


def _tc_scale(xw, deg):
    return pl.pallas_call(
        _scale_body,
        grid=(_NPAD // _BR,),
        in_specs=[
            pl.BlockSpec((_BR, _D), lambda i: (i, 0)),
            pl.BlockSpec((_BR, 1), lambda i: (i, 0)),
        ],
        out_specs=[
            pl.BlockSpec((_BR, _D), lambda i: (i, 0)),
            pl.BlockSpec((_BR, 1), lambda i: (i, 0)),
        ],
        out_shape=[
            jax.ShapeDtypeStruct((_NPAD, _D), jnp.float32),
            jax.ShapeDtypeStruct((_NPAD, 1), jnp.float32),
        ],
    )(xw, deg)


def _mid_body(a_ref, dinv_ref, b1_ref, w2_ref, y2_ref):
    dinv = dinv_ref[...]
    h1 = jnp.maximum((a_ref[0] + a_ref[1]) * dinv + b1_ref[...], 0.0)
    y2_ref[...] = jnp.dot(h1, w2_ref[...],
                          preferred_element_type=jnp.float32) * dinv


def _tc_mid(a, dinv, b1, W2):
    return pl.pallas_call(
        _mid_body,
        grid=(_NPAD // _BR,),
        in_specs=[
            pl.BlockSpec((2, _BR, _D), lambda i: (0, i, 0)),
            pl.BlockSpec((_BR, 1), lambda i: (i, 0)),
            pl.BlockSpec((1, _D), lambda i: (0, 0)),
            pl.BlockSpec((_D, _D), lambda i: (0, 0)),
        ],
        out_specs=pl.BlockSpec((_BR, _D), lambda i: (i, 0)),
        out_shape=jax.ShapeDtypeStruct((_NPAD, _D), jnp.float32),
    )(a, dinv, b1, W2)


def _fin_body(a_ref, dinv_ref, b2_ref, p1w_ref, p1b_ref,
              p2w_ref, p2b_ref, z_ref, h_ref):
    dinv = dinv_ref[...]
    z = (a_ref[0] + a_ref[1]) * dinv + b2_ref[...]
    z_ref[...] = z
    t = jnp.maximum(
        jnp.dot(z, p1w_ref[...], preferred_element_type=jnp.float32)
        + p1b_ref[...], 0.0)
    h_ref[...] = jnp.dot(t, p2w_ref[...],
                         preferred_element_type=jnp.float32) + p2b_ref[...]


def _tc_fin(a, dinv, b2, P1w, P1b, P2w, P2b):
    return pl.pallas_call(
        _fin_body,
        grid=(_NPAD // _BR,),
        in_specs=[
            pl.BlockSpec((2, _BR, _D), lambda i: (0, i, 0)),
            pl.BlockSpec((_BR, 1), lambda i: (i, 0)),
            pl.BlockSpec((1, _D), lambda i: (0, 0)),
            pl.BlockSpec((_D, _D), lambda i: (0, 0)),
            pl.BlockSpec((1, _D), lambda i: (0, 0)),
            pl.BlockSpec((_D, _DP), lambda i: (0, 0)),
            pl.BlockSpec((1, _DP), lambda i: (0, 0)),
        ],
        out_specs=[
            pl.BlockSpec((_BR, _D), lambda i: (i, 0)),
            pl.BlockSpec((_BR, _DP), lambda i: (i, 0)),
        ],
        out_shape=[
            jax.ShapeDtypeStruct((_N, _D), jnp.float32),
            jax.ShapeDtypeStruct((_N, _DP), jnp.float32),
        ],
    )(a, dinv, b2, P1w, P1b, P2w, P2b)


def kernel(x, edge_index, W1, b1, W2, b2, P1w, P1b, P2w, P2b):
    src = edge_index[0]
    dst = edge_index[1]
    dstp = jnp.concatenate([dst, _TAIL_DST]).reshape(32, _NCHT, _CH)
    srcp = jnp.concatenate([src, _TAIL_SRC]).reshape(32, _NCHT, _CH)

    xpad = jnp.zeros((_NPAD, _D), jnp.float32).at[:_N].set(x)
    zeros1 = jnp.zeros((_NPAD,), jnp.float32)
    zeros2 = jnp.zeros((_NPAD, _D), jnp.float32)

    degp = _deg_sc(dstp, zeros1)
    deg = (degp[:_NPAD] + degp[_NPAD:]).reshape(_NPAD, 1)
    xw1 = _tc_xw(xpad, W1)
    y1, dinv = _tc_scale(xw1, deg)

    agg1 = _agg_sc(y1, srcp, dstp, zeros2).reshape(_NC, _NPAD, _D)
    y2 = _tc_mid(agg1, dinv, b1.reshape(1, _D), W2)

    agg2 = _agg_sc(y2, srcp, dstp, zeros2).reshape(_NC, _NPAD, _D)
    z, h = _tc_fin(agg2, dinv, b2.reshape(1, _D),
                   P1w, P1b.reshape(1, _D), P2w, P2b.reshape(1, _DP))
    return (z, h)
